# Initial kernel scaffold; baseline (speedup 1.0000x reference)
import jax
import jax.numpy as jnp
from jax.experimental import pallas as pl
from jax.experimental.pallas import tpu as pltpu

DIM = 512
CB = 256
NCB = 16
DPC = 32
TBLK = 1024


def _vq_kernel(x_ref, a_ref, b_ref, out_ref, acc_ref):
    i = pl.program_id(0)

    @pl.when(i == 0)
    def _init():
        acc_ref[0] = 0.0
        acc_ref[1] = 0.0

    p_tot = jnp.float32(0.0)
    s_tot = jnp.float32(0.0)
    for c in range(NCB):
        xc = x_ref[:, c * DPC:(c + 1) * DPC]
        ac = a_ref[c * DPC:(c + 1) * DPC, :]
        prod = jnp.dot(xc.astype(jnp.bfloat16), ac,
                       preferred_element_type=jnp.float32)
        logits = prod[:, :CB] + b_ref[c:c + 1, :]
        cross = prod[:, CB:]
        tt = ac[:, CB:].astype(jnp.float32)
        nrm = jnp.sum(tt * tt, axis=0, keepdims=True)
        m = jnp.max(logits, axis=1, keepdims=True)
        val = jnp.where(logits == m, nrm - 2.0 * cross, 0.0)
        p_tot += jnp.sum(val)
        s_tot += jnp.sum(xc * xc)

    acc_ref[0] += p_tot
    acc_ref[1] += s_tot

    @pl.when(i == pl.num_programs(0) - 1)
    def _fin():
        s = acc_ref[1] + 1e-20
        out_ref[0, 0] = (acc_ref[0] + acc_ref[1]) / s


def kernel(x, W, b, to_output, mask):
    del mask
    n_tokens = x.shape[0]

    w4 = W.reshape(NCB, CB, NCB, DPC)
    t4 = to_output.reshape(NCB, CB, NCB, DPC)
    diag = jnp.arange(NCB)
    wblk = w4[diag, :, diag, :]
    tblk = t4[diag, :, diag, :]
    a = jnp.concatenate(
        [jnp.transpose(wblk, (0, 2, 1)).reshape(NCB * DPC, CB),
         jnp.transpose(tblk, (0, 2, 1)).reshape(NCB * DPC, CB)],
        axis=1).astype(jnp.bfloat16)
    b2 = b.reshape(NCB, CB)

    grid = n_tokens // TBLK
    out = pl.pallas_call(
        _vq_kernel,
        grid=(grid,),
        in_specs=[
            pl.BlockSpec((TBLK, DIM), lambda i: (i, 0)),
            pl.BlockSpec((NCB * DPC, 2 * CB), lambda i: (0, 0)),
            pl.BlockSpec((NCB, CB), lambda i: (0, 0)),
        ],
        out_specs=pl.BlockSpec((1, 1), lambda i: (0, 0)),
        out_shape=jax.ShapeDtypeStruct((1, 1), jnp.float32),
        scratch_shapes=[pltpu.SMEM((2,), jnp.float32)],
    )(x, a, b2)
    return out.reshape(())


# speedup vs baseline: 24006.7993x; 24006.7993x over previous
import jax
import jax.numpy as jnp
from jax.experimental import pallas as pl
from jax.experimental.pallas import tpu as pltpu

DIM = 512
CB = 256
NCB = 16
DPC = 32
TBLK = 1024


def _vq_kernel(x_ref, a_ref, b_ref, out_ref, acc_ref):
    i = pl.program_id(0)

    @pl.when(i == 0)
    def _init():
        acc_ref[0] = 0.0
        acc_ref[1] = 0.0

    p_tot = jnp.float32(0.0)
    s_tot = jnp.float32(0.0)
    for c in range(NCB):
        xc = x_ref[:, c * DPC:(c + 1) * DPC]
        ac = a_ref[c * DPC:(c + 1) * DPC, :]
        prod = jnp.dot(xc.astype(jnp.bfloat16), ac,
                       preferred_element_type=jnp.float32)
        logits = prod[:, :CB] + b_ref[c:c + 1, :]
        cross = prod[:, CB:]
        tt = ac[:, CB:].astype(jnp.float32)
        nrm = jnp.sum(tt * tt, axis=0, keepdims=True)
        m = jnp.max(logits, axis=1, keepdims=True)
        val = jnp.where(logits == m, nrm - 2.0 * cross, 0.0)
        p_tot += jnp.sum(val)
        s_tot += jnp.sum(xc * xc)

    acc_ref[0] += p_tot
    acc_ref[1] += s_tot

    @pl.when(i == pl.num_programs(0) - 1)
    def _fin():
        s = acc_ref[1] + 1e-20
        out_ref[...] = jnp.full((1, 1), (acc_ref[0] + acc_ref[1]) / s,
                                dtype=jnp.float32)


def kernel(x, W, b, to_output, mask):
    del mask
    n_tokens = x.shape[0]

    w4 = W.reshape(NCB, CB, NCB, DPC)
    t4 = to_output.reshape(NCB, CB, NCB, DPC)
    diag = jnp.arange(NCB)
    wblk = w4[diag, :, diag, :]
    tblk = t4[diag, :, diag, :]
    a = jnp.concatenate(
        [jnp.transpose(wblk, (0, 2, 1)).reshape(NCB * DPC, CB),
         jnp.transpose(tblk, (0, 2, 1)).reshape(NCB * DPC, CB)],
        axis=1).astype(jnp.bfloat16)
    b2 = b.reshape(NCB, CB)

    grid = n_tokens // TBLK
    out = pl.pallas_call(
        _vq_kernel,
        grid=(grid,),
        in_specs=[
            pl.BlockSpec((TBLK, DIM), lambda i: (i, 0)),
            pl.BlockSpec((NCB * DPC, 2 * CB), lambda i: (0, 0)),
            pl.BlockSpec((NCB, CB), lambda i: (0, 0)),
        ],
        out_specs=pl.BlockSpec((1, 1), lambda i: (0, 0)),
        out_shape=jax.ShapeDtypeStruct((1, 1), jnp.float32),
        scratch_shapes=[pltpu.SMEM((2,), jnp.float32)],
    )(x, a, b2)
    return out.reshape(())


# speedup vs baseline: 41355.9415x; 1.7227x over previous
import jax
import jax.numpy as jnp
from jax.experimental import pallas as pl
from jax.experimental.pallas import tpu as pltpu

DIM = 512
CB = 256
NCB = 16
DPC = 32
TBLK = 1024


def _vq_kernel(x_ref, a_ref, b_ref, out_ref, nrm_ref, acc_ref):
    i = pl.program_id(0)

    @pl.when(i == 0)
    def _init():
        acc_ref[0] = 0.0
        acc_ref[1] = 0.0
        for c in range(NCB):
            tt = a_ref[c * DPC:(c + 1) * DPC, CB:].astype(jnp.float32)
            nrm_ref[c:c + 1, :] = 0.25 * jnp.sum(tt * tt, axis=0,
                                                 keepdims=True)

    xall = x_ref[...]
    xbf = xall.astype(jnp.bfloat16)
    vacc = jnp.zeros((TBLK, CB), jnp.float32)
    for c in range(NCB):
        ac = a_ref[c * DPC:(c + 1) * DPC, :]
        prod = jnp.dot(xbf[:, c * DPC:(c + 1) * DPC], ac,
                       preferred_element_type=jnp.float32)
        logits = prod[:, :CB] + b_ref[c:c + 1, :]
        m = jnp.max(logits, axis=1, keepdims=True)
        fval = prod[:, CB:] + nrm_ref[c:c + 1, :]
        vacc += jnp.where(logits == m, fval, 0.0)
    acc_ref[0] += jnp.sum(vacc)
    acc_ref[1] += jnp.sum(xall * xall)

    @pl.when(i == pl.num_programs(0) - 1)
    def _fin():
        s = acc_ref[1] + 1e-20
        out_ref[...] = jnp.full((1, 1), (acc_ref[0] + acc_ref[1]) / s,
                                dtype=jnp.float32)


def kernel(x, W, b, to_output, mask):
    del mask
    n_tokens = x.shape[0]

    w4 = W.reshape(NCB, CB, NCB, DPC)
    t4 = to_output.reshape(NCB, CB, NCB, DPC)
    diag = jnp.arange(NCB)
    wblk = w4[diag, :, diag, :]
    tblk = t4[diag, :, diag, :]
    a = jnp.concatenate(
        [jnp.transpose(wblk, (0, 2, 1)).reshape(NCB * DPC, CB),
         -2.0 * jnp.transpose(tblk, (0, 2, 1)).reshape(NCB * DPC, CB)],
        axis=1).astype(jnp.bfloat16)
    b2 = b.reshape(NCB, CB)

    grid = n_tokens // TBLK
    out = pl.pallas_call(
        _vq_kernel,
        grid=(grid,),
        in_specs=[
            pl.BlockSpec((TBLK, DIM), lambda i: (i, 0)),
            pl.BlockSpec((NCB * DPC, 2 * CB), lambda i: (0, 0)),
            pl.BlockSpec((NCB, CB), lambda i: (0, 0)),
        ],
        out_specs=pl.BlockSpec((1, 1), lambda i: (0, 0)),
        out_shape=jax.ShapeDtypeStruct((1, 1), jnp.float32),
        scratch_shapes=[pltpu.VMEM((NCB, CB), jnp.float32),
                        pltpu.SMEM((2,), jnp.float32)],
    )(x, a, b2)
    return out.reshape(())
